# Initial kernel scaffold; baseline (speedup 1.0000x reference)
import jax
import jax.numpy as jnp
from jax.experimental import pallas as pl


def kernel(feature_a, feature_b, edgelist_a_b_pos, edgelist_a_b_neg, edgelist_a_a_pos, edgelist_a_a_neg, edgelist_b_a_pos, edgelist_b_a_neg, edgelist_b_b_pos, edgelist_b_b_neg, W_agg, b_agg, W_u1, b_u1, prelu_a, W_u2, b_u2):
    raise NotImplementedError("write your pallas kernel here")


# speedup vs baseline: 14.1970x; 14.1970x over previous
import functools

import jax
import jax.numpy as jnp
from jax import lax
from jax.experimental import pallas as pl
from jax.experimental.pallas import tpu as pltpu
from jax.experimental.pallas import tpu_sc as plsc

N = 50000
D = 32
E = 800000
NLISTS = 8

NC = 2
NS = 16
B = 128
G = 8
CPT = 49
BPT = G * CPT
EPT = BPT * B
E_PAD = EPT * NS
NB = E_PAD // B

ACC_N = 50176
RPT = ACC_N // NS
ZR = 112
ZD = 784

R = 2000
NBLK = N // R


def _sc_agg(feat_hbm, src_hbm, dst_hbm, acc_out, deg_out,
            acc_sp, deg_sp, src_buf, dst_buf, rows0, rows1, ones_v,
            zbuf, zdbuf, sem0, sem1):
    c = lax.axis_index("c")
    s = lax.axis_index("s")
    zero16 = jnp.zeros((16,), jnp.float32)

    def _zrow(r, _):
        zbuf[r, pl.ds(0, 16)] = zero16
        zbuf[r, pl.ds(16, 16)] = zero16
        return _
    lax.fori_loop(0, ZR, _zrow, None)

    def _zdeg(i, _):
        zdbuf[pl.ds(i * 16, 16)] = zero16
        return _
    lax.fori_loop(0, ZD // 16, _zdeg, None)

    one16 = jnp.ones((16,), jnp.float32)
    for j in range(B // 16):
        ones_v[pl.ds(j * 16, 16)] = one16

    row0 = s * RPT
    rowsb = (rows0, rows1)
    semsb = (sem0, sem1)

    for li in range(4):
        l = c * 4 + li

        for z in range(RPT // ZR):
            pltpu.sync_copy(zbuf, acc_sp.at[pl.ds(row0 + z * ZR, ZR), :])
        for z in range(RPT // ZD):
            pltpu.sync_copy(zdbuf, deg_sp.at[pl.ds(row0 + z * ZD, ZD)])
        plsc.subcore_barrier()

        def _chunk(ch, _):
            base = s * BPT + ch * G
            pltpu.sync_copy(src_hbm.at[l, pl.ds(base, G)], src_buf)
            pltpu.sync_copy(dst_hbm.at[l, pl.ds(base, G)], dst_buf)
            desc = pltpu.async_copy(feat_hbm.at[src_buf.at[0]], rows0, sem0)
            for j in range(G):
                nxt = None
                if j + 1 < G:
                    nxt = pltpu.async_copy(
                        feat_hbm.at[src_buf.at[j + 1]],
                        rowsb[(j + 1) % 2], semsb[(j + 1) % 2])
                desc.wait()
                pltpu.sync_copy(rowsb[j % 2], acc_sp.at[dst_buf.at[j]],
                                add=True)
                pltpu.sync_copy(ones_v, deg_sp.at[dst_buf.at[j]], add=True)
                desc = nxt
            return _
        lax.fori_loop(0, CPT, _chunk, None)
        plsc.subcore_barrier()

        pltpu.sync_copy(acc_sp.at[pl.ds(row0, RPT), :],
                        acc_out.at[l, pl.ds(row0, RPT), :])
        pltpu.sync_copy(deg_sp.at[pl.ds(row0, RPT)],
                        deg_out.at[pl.ds(l * ACC_N + row0, RPT)])


_sc_agg_call = functools.partial(
    pl.kernel,
    out_type=(jax.ShapeDtypeStruct((NLISTS, ACC_N, D), jnp.float32),
              jax.ShapeDtypeStruct((NLISTS * ACC_N,), jnp.float32)),
    mesh=plsc.VectorSubcoreMesh(core_axis_name="c", subcore_axis_name="s",
                                num_cores=NC, num_subcores=NS),
    compiler_params=pltpu.CompilerParams(use_tc_tiling_on_sc=False),
    scratch_types=[
        pltpu.VMEM_SHARED((ACC_N, D), jnp.float32),
        pltpu.VMEM_SHARED((ACC_N,), jnp.float32),
        pltpu.VMEM((G, B), jnp.int32),
        pltpu.VMEM((G, B), jnp.int32),
        pltpu.VMEM((B, D), jnp.float32),
        pltpu.VMEM((B, D), jnp.float32),
        pltpu.VMEM((B,), jnp.float32),
        pltpu.VMEM((ZR, D), jnp.float32),
        pltpu.VMEM((ZD,), jnp.float32),
        pltpu.SemaphoreType.DMA,
        pltpu.SemaphoreType.DMA,
    ],
)(_sc_agg)


def _tc_update(feat_ref, acc_ref, deg_ref, wagg_ref, bagg_ref,
               w1_ref, b1_ref, w2_ref, b2_ref, alpha_ref, out_ref):
    degs = deg_ref[...]
    degs = jnp.where(degs == 0.0, 1.0, degs)
    means = acc_ref[...] / degs
    dn = (((1,), (1,)), ((), ()))
    ms = [lax.dot_general(means[i], wagg_ref[i], dn,
                          preferred_element_type=jnp.float32)
          + bagg_ref[i][None, :] for i in range(4)]
    h = jnp.concatenate([feat_ref[...]] + ms, axis=1)
    u = lax.dot_general(h, w1_ref[...], dn,
                        preferred_element_type=jnp.float32) + b1_ref[...]
    a = alpha_ref[0, 0]
    u = jnp.where(u >= 0.0, u, a * u)
    out_ref[...] = lax.dot_general(u, w2_ref[...], dn,
                                   preferred_element_type=jnp.float32) \
        + b2_ref[...]


def _tc_call(feat_cat, acc, deg3, W_agg, b_agg, W_u1, b1, W_u2, b2, alpha):
    return pl.pallas_call(
        _tc_update,
        grid=(2, NBLK),
        in_specs=[
            pl.BlockSpec((R, D), lambda t, j: (t * NBLK + j, 0)),
            pl.BlockSpec((4, R, D), lambda t, j: (t, j, 0)),
            pl.BlockSpec((4, R, 1), lambda t, j: (t, j, 0)),
            pl.BlockSpec((4, D, D), lambda t, j: (t, 0, 0)),
            pl.BlockSpec((None, 4, D), lambda t, j: (t, 0, 0)),
            pl.BlockSpec((2 * D, 5 * D), lambda t, j: (0, 0)),
            pl.BlockSpec((1, 2 * D), lambda t, j: (0, 0)),
            pl.BlockSpec((D, 2 * D), lambda t, j: (0, 0)),
            pl.BlockSpec((1, D), lambda t, j: (0, 0)),
            pl.BlockSpec(memory_space=pltpu.SMEM),
        ],
        out_specs=pl.BlockSpec((None, R, D), lambda t, j: (t, j, 0)),
        out_shape=jax.ShapeDtypeStruct((2, N, D), jnp.float32),
    )(feat_cat, acc, deg3, W_agg, b_agg, W_u1, b1, W_u2, b2, alpha)


def kernel(feature_a, feature_b,
           edgelist_a_b_pos, edgelist_a_b_neg, edgelist_a_a_pos,
           edgelist_a_a_neg, edgelist_b_a_pos, edgelist_b_a_neg,
           edgelist_b_b_pos, edgelist_b_b_neg,
           W_agg, b_agg, W_u1, b_u1, prelu_a, W_u2, b_u2):
    feat_cat = jnp.concatenate([feature_a, feature_b], axis=0)

    edge_lists = (edgelist_a_b_pos, edgelist_a_b_neg, edgelist_a_a_pos,
                  edgelist_a_a_neg, edgelist_b_a_pos, edgelist_b_a_neg,
                  edgelist_b_b_pos, edgelist_b_b_neg)
    offs = (N, N, 0, 0, 0, 0, N, N)

    npad = E_PAD - E
    pad_src = (jnp.arange(npad, dtype=jnp.int32) % N)
    pad_dst = N + (jnp.arange(npad, dtype=jnp.int32) % (ACC_N - N))
    srcs, dsts = [], []
    for e, off in zip(edge_lists, offs):
        srcs.append(jnp.concatenate([e[:, 1] + off, pad_src]))
        dsts.append(jnp.concatenate([e[:, 0], pad_dst]))
    src_all = jnp.stack(srcs).reshape(NLISTS, NB, B)
    dst_all = jnp.stack(dsts).reshape(NLISTS, NB, B)

    acc, deg = _sc_agg_call(feat_cat, src_all, dst_all)
    deg3 = deg.reshape(NLISTS, ACC_N, 1)

    out = _tc_call(feat_cat, acc, deg3, W_agg, b_agg.reshape(2, 4, D),
                   W_u1, b_u1.reshape(1, 2 * D), W_u2,
                   b_u2.reshape(1, D), prelu_a.reshape(1, 1))
    return (out[0], out[1])


# speedup vs baseline: 18.3806x; 1.2947x over previous
import jax
import jax.numpy as jnp
from jax import lax
from jax.experimental import pallas as pl
from jax.experimental.pallas import tpu as pltpu
from jax.experimental.pallas import tpu_sc as plsc

N = 50000
D = 32
E = 800000
NLISTS = 8

NC = 2
NS = 16
B = 128
CH = 8
MAC = 25
NPROC = 2 * MAC
NCHT = NPROC + 1
EPT = NPROC * CH * B
E_PAD = EPT * NS
NBT = NCHT * CH

ACC_N = 50176
RPT = ACC_N // NS

R = 2000
NBLK = N // R


def _sc_agg(feat_hbm, idx_hbm, zrows_hbm, zdeg_hbm, acc_out, deg_out,
            acc_sp, deg_sp, bufA, bufB, r0, r1, r2, r3, ones_v,
            gs0, gs1, gs2, gs3, ss0, ss1, ss2, ss3, dsem, isA, isB):
    c = lax.axis_index("c")
    s = lax.axis_index("s")
    rows = (r0, r1, r2, r3)
    gsem = (gs0, gs1, gs2, gs3)
    ssem = (ss0, ss1, ss2, ss3)

    one16 = jnp.ones((16,), jnp.float32)
    for j in range(B // 16):
        ones_v[pl.ds(j * 16, 16)] = one16

    row0 = s * RPT
    tb = s * NBT

    for li in range(4):
        l = c * 4 + li

        pltpu.sync_copy(zrows_hbm.at[pl.ds(row0, RPT), :],
                        acc_sp.at[pl.ds(row0, RPT), :])
        pltpu.sync_copy(zdeg_hbm.at[pl.ds(row0, RPT)],
                        deg_sp.at[pl.ds(row0, RPT)])
        plsc.subcore_barrier()

        pltpu.async_copy(idx_hbm.at[l, pl.ds(tb, CH)], bufA, isA).wait()
        pltpu.async_copy(feat_hbm.at[bufA.at[0, 1]], rows[0], gsem[0])
        pltpu.async_copy(feat_hbm.at[bufA.at[1, 1]], rows[1], gsem[1])

        def _macro(m, _):
            for u in range(16):
                cur = u % 4
                nx = (u + 2) % 4
                cbuf, crow = (bufA, u) if u < 8 else (bufB, u - 8)
                v = u + 2
                if v < 8:
                    nbuf, nrow = bufA, v
                elif v < 16:
                    nbuf, nrow = bufB, v - 8
                else:
                    nbuf, nrow = bufA, v - 16

                if u == 2:
                    pltpu.async_copy(
                        idx_hbm.at[l, pl.ds(tb + CH * (2 * m + 1), CH)],
                        bufB, isB)
                if u == 5:
                    pltpu.make_async_copy(
                        idx_hbm.at[l, pl.ds(tb + CH * (2 * m + 1), CH)],
                        bufB, isB).wait()
                if u == 10:
                    pltpu.async_copy(
                        idx_hbm.at[l, pl.ds(tb + CH * (2 * m + 2), CH)],
                        bufA, isA)
                if u == 13:
                    pltpu.make_async_copy(
                        idx_hbm.at[l, pl.ds(tb + CH * (2 * m + 2), CH)],
                        bufA, isA).wait()

                def _drain_sc():
                    pltpu.make_async_copy(
                        rows[nx], acc_sp.at[cbuf.at[crow, 0]],
                        ssem[nx]).wait()

                def _drain_dg():
                    pltpu.make_async_copy(
                        ones_v, deg_sp.at[cbuf.at[crow, 0]], dsem).wait()

                if u < 2:
                    @pl.when(m > 0)
                    def _():
                        _drain_sc()
                        _drain_dg()
                else:
                    _drain_sc()
                    _drain_dg()

                pltpu.async_copy(feat_hbm.at[nbuf.at[nrow, 1]],
                                 rows[nx], gsem[nx])
                pltpu.make_async_copy(feat_hbm.at[cbuf.at[crow, 1]],
                                      rows[cur], gsem[cur]).wait()
                pltpu.async_copy(rows[cur], acc_sp.at[cbuf.at[crow, 0]],
                                 ssem[cur], add=True)
                pltpu.async_copy(ones_v, deg_sp.at[cbuf.at[crow, 0]],
                                 dsem, add=True)
            return _
        lax.fori_loop(0, MAC, _macro, None)

        pltpu.make_async_copy(feat_hbm.at[bufA.at[0, 1]], rows[0],
                              gsem[0]).wait()
        pltpu.make_async_copy(feat_hbm.at[bufA.at[1, 1]], rows[1],
                              gsem[1]).wait()
        for x in (2, 3):
            pltpu.make_async_copy(rows[x], acc_sp.at[bufB.at[x + 4, 0]],
                                  ssem[x]).wait()
        for _x in range(2):
            pltpu.make_async_copy(ones_v, deg_sp.at[bufB.at[6, 0]],
                                  dsem).wait()
        plsc.subcore_barrier()

        pltpu.sync_copy(acc_sp.at[pl.ds(row0, RPT), :],
                        acc_out.at[l, pl.ds(row0, RPT), :])
        pltpu.sync_copy(deg_sp.at[pl.ds(row0, RPT)],
                        deg_out.at[pl.ds(l * ACC_N + row0, RPT)])


_sc_agg_call = pl.kernel(
    _sc_agg,
    out_type=(jax.ShapeDtypeStruct((NLISTS, ACC_N, D), jnp.float32),
              jax.ShapeDtypeStruct((NLISTS * ACC_N,), jnp.float32)),
    mesh=plsc.VectorSubcoreMesh(core_axis_name="c", subcore_axis_name="s",
                                num_cores=NC, num_subcores=NS),
    compiler_params=pltpu.CompilerParams(use_tc_tiling_on_sc=False),
    scratch_types=[
        pltpu.VMEM_SHARED((ACC_N, D), jnp.float32),
        pltpu.VMEM_SHARED((ACC_N,), jnp.float32),
        pltpu.VMEM((CH, 2, B), jnp.int32),
        pltpu.VMEM((CH, 2, B), jnp.int32),
        pltpu.VMEM((B, D), jnp.float32),
        pltpu.VMEM((B, D), jnp.float32),
        pltpu.VMEM((B, D), jnp.float32),
        pltpu.VMEM((B, D), jnp.float32),
        pltpu.VMEM((B,), jnp.float32),
    ] + [pltpu.SemaphoreType.DMA] * 11,
)


def _tc_update(feat_ref, acc_ref, deg_ref, wagg_ref, bagg_ref,
               w1_ref, b1_ref, w2_ref, b2_ref, alpha_ref, out_ref):
    degs = deg_ref[...]
    degs = jnp.where(degs == 0.0, 1.0, degs)
    means = acc_ref[...] / degs
    dn = (((1,), (1,)), ((), ()))
    ms = [lax.dot_general(means[i], wagg_ref[i], dn,
                          preferred_element_type=jnp.float32)
          + bagg_ref[i][None, :] for i in range(4)]
    h = jnp.concatenate([feat_ref[...]] + ms, axis=1)
    u = lax.dot_general(h, w1_ref[...], dn,
                        preferred_element_type=jnp.float32) + b1_ref[...]
    a = alpha_ref[0, 0]
    u = jnp.where(u >= 0.0, u, a * u)
    out_ref[...] = lax.dot_general(u, w2_ref[...], dn,
                                   preferred_element_type=jnp.float32) \
        + b2_ref[...]


def _tc_call(feat_cat, acc, deg3, W_agg, b_agg, W_u1, b1, W_u2, b2, alpha):
    return pl.pallas_call(
        _tc_update,
        grid=(2, NBLK),
        in_specs=[
            pl.BlockSpec((R, D), lambda t, j: (t * NBLK + j, 0)),
            pl.BlockSpec((4, R, D), lambda t, j: (t, j, 0)),
            pl.BlockSpec((4, R, 1), lambda t, j: (t, j, 0)),
            pl.BlockSpec((4, D, D), lambda t, j: (t, 0, 0)),
            pl.BlockSpec((None, 4, D), lambda t, j: (t, 0, 0)),
            pl.BlockSpec((2 * D, 5 * D), lambda t, j: (0, 0)),
            pl.BlockSpec((1, 2 * D), lambda t, j: (0, 0)),
            pl.BlockSpec((D, 2 * D), lambda t, j: (0, 0)),
            pl.BlockSpec((1, D), lambda t, j: (0, 0)),
            pl.BlockSpec(memory_space=pltpu.SMEM),
        ],
        out_specs=pl.BlockSpec((None, R, D), lambda t, j: (t, j, 0)),
        out_shape=jax.ShapeDtypeStruct((2, N, D), jnp.float32),
    )(feat_cat, acc, deg3, W_agg, b_agg, W_u1, b1, W_u2, b2, alpha)


def kernel(feature_a, feature_b,
           edgelist_a_b_pos, edgelist_a_b_neg, edgelist_a_a_pos,
           edgelist_a_a_neg, edgelist_b_a_pos, edgelist_b_a_neg,
           edgelist_b_b_pos, edgelist_b_b_neg,
           W_agg, b_agg, W_u1, b_u1, prelu_a, W_u2, b_u2):
    feat_cat = jnp.concatenate([feature_a, feature_b], axis=0)

    edge_lists = (edgelist_a_b_pos, edgelist_a_b_neg, edgelist_a_a_pos,
                  edgelist_a_a_neg, edgelist_b_a_pos, edgelist_b_a_neg,
                  edgelist_b_b_pos, edgelist_b_b_neg)
    offs = (N, N, 0, 0, 0, 0, N, N)

    npad = E_PAD - E
    pad_src = jnp.arange(npad, dtype=jnp.int32) % N
    pad_dst = N + jnp.arange(npad, dtype=jnp.int32) % (ACC_N - N)
    ndum = NS * CH * B
    dum_src = (jnp.arange(ndum, dtype=jnp.int32) % N).reshape(NS, CH * B)
    dum_dst = (N + jnp.arange(ndum, dtype=jnp.int32)
               % (ACC_N - N)).reshape(NS, CH * B)

    idxs = []
    for e, off in zip(edge_lists, offs):
        srcp = jnp.concatenate([e[:, 1] + off, pad_src]).reshape(NS, EPT)
        dstp = jnp.concatenate([e[:, 0], pad_dst]).reshape(NS, EPT)
        srcf = jnp.concatenate([srcp, dum_src], axis=1).reshape(-1, B)
        dstf = jnp.concatenate([dstp, dum_dst], axis=1).reshape(-1, B)
        idxs.append(jnp.stack([dstf, srcf], axis=1))
    idx_all = jnp.stack(idxs)

    zrows = jnp.zeros((ACC_N, D), jnp.float32)
    zdeg = jnp.zeros((ACC_N,), jnp.float32)

    acc, deg = _sc_agg_call(feat_cat, idx_all, zrows, zdeg)
    deg3 = deg.reshape(NLISTS, ACC_N, 1)

    out = _tc_call(feat_cat, acc, deg3, W_agg, b_agg.reshape(2, 4, D),
                   W_u1, b_u1.reshape(1, 2 * D), W_u2,
                   b_u2.reshape(1, D), prelu_a.reshape(1, 1))
    return (out[0], out[1])
